# baseline (speedup 1.0000x reference)
import functools

import jax
import jax.numpy as jnp
from jax import lax
from jax.experimental import pallas as pl
from jax.experimental.pallas import tpu as pltpu
from jax.experimental.pallas import tpu_sc as plsc

N = 10000
E = 320000
D = 128
L = 3
G = 64

NC = 2
NS = 16
NW = NC * NS
EB = 96
BPT = 106
EPAD = NW * BPT * EB
RPT = 632
NPAD = NS * RPT


def _edge_body(h_hbm, t_hbm, src_hbm, cid_hbm, dst_hbm, zero_hbm, out_hbm,
               isa, isb, ica, icb, idsa, idsb, ida, idb,
               hba, hbb, eba, ebb,
               sia, sib, sga, sgb, ssa, ssb, aggr):
    cid = lax.axis_index("c")
    sid = lax.axis_index("s")
    wid = sid * NC + cid
    isx = (isa, isb)
    icx = (ica, icb)
    idsx = (idsa, idsb)
    idd = (ida, idb)
    hb = (hba, hbb)
    ebf = (eba, ebb)
    si = (sia, sib)
    sg = (sga, sgb)
    ss = (ssa, ssb)

    pltpu.sync_copy(zero_hbm.at[pl.ds(sid * RPT, RPT)],
                    aggr.at[pl.ds(sid * RPT, RPT)])
    plsc.subcore_barrier()

    tb = wid * BPT

    def issue_idx(b, p):
        base = (tb + b) * EB
        pltpu.async_copy(src_hbm.at[pl.ds(base, EB)], isx[p], si[p])
        pltpu.async_copy(cid_hbm.at[pl.ds(base, EB)], icx[p], si[p])
        pltpu.async_copy(dst_hbm.at[pl.ds(base, EB)], idsx[p], si[p])

    def wait_idx(b, p):
        base = (tb + b) * EB
        pltpu.make_async_copy(src_hbm.at[pl.ds(base, EB)], isx[p], si[p]).wait()
        pltpu.make_async_copy(cid_hbm.at[pl.ds(base, EB)], icx[p], si[p]).wait()
        pltpu.make_async_copy(dst_hbm.at[pl.ds(base, EB)], idsx[p], si[p]).wait()

    def gathers(p):
        pltpu.async_copy(h_hbm.at[isx[p]], hb[p], sg[p])
        pltpu.async_copy(t_hbm.at[icx[p]], ebf[p], sg[p])

    def wait_gathers(p):
        pltpu.make_async_copy(h_hbm.at[isx[p]], hb[p], sg[p]).wait()
        pltpu.make_async_copy(t_hbm.at[icx[p]], ebf[p], sg[p]).wait()

    def copy_dst_idx(p):
        for j in range(EB // 16):
            sl = pl.ds(j * 16, 16)
            idd[p][sl] = idsx[p][sl]

    def compute(p):
        def row(i, c2):
            for j in range(D // 16):
                sl = pl.ds(j * 16, 16)
                hb[p][i, sl] = jnp.maximum(hb[p][i, sl] + ebf[p][i, sl], 0.0)
            return c2
        lax.fori_loop(0, EB, row, 0)

    def scatter(p):
        pltpu.async_copy(hb[p], aggr.at[idd[p]], ss[p], add=True)

    def wait_scatter(p):
        pltpu.make_async_copy(hb[p], aggr.at[idd[p]], ss[p]).wait()

    issue_idx(0, 0)
    issue_idx(1, 1)
    wait_idx(0, 0)
    gathers(0)

    wait_gathers(0)
    copy_dst_idx(0)
    wait_idx(1, 1)
    gathers(1)
    issue_idx(2, 0)
    compute(0)
    scatter(0)

    wait_gathers(1)
    copy_dst_idx(1)
    wait_idx(2, 0)
    wait_scatter(0)
    gathers(0)
    issue_idx(3, 1)
    compute(1)
    scatter(1)

    def pair(j, carry):
        for k in range(2):
            b = 2 * j + k
            q = 1 - k
            wait_gathers(k)
            copy_dst_idx(k)
            wait_idx(b + 1, q)
            wait_scatter(q)
            gathers(q)
            issue_idx(b + 2, k)
            compute(k)
            scatter(k)
        return carry

    lax.fori_loop(1, BPT // 2 - 1, pair, 0)

    wait_gathers(0)
    copy_dst_idx(0)
    wait_idx(BPT - 1, 1)
    wait_scatter(1)
    gathers(1)
    compute(0)
    scatter(0)

    wait_gathers(1)
    copy_dst_idx(1)
    compute(1)
    wait_scatter(0)
    scatter(1)
    wait_scatter(1)

    plsc.subcore_barrier()
    pltpu.sync_copy(aggr.at[pl.ds(sid * RPT, RPT)],
                    out_hbm.at[cid, pl.ds(sid * RPT, RPT)])


@functools.lru_cache(maxsize=None)
def _edge_kernel():
    return _edge_wrap(_edge_body)


def _edge_wrap(body):
    return pl.kernel(
        body,
        out_type=jax.ShapeDtypeStruct((NC, NPAD, D), jnp.float32),
        mesh=plsc.VectorSubcoreMesh(core_axis_name="c", subcore_axis_name="s"),
        scratch_types=[
            pltpu.VMEM((EB,), jnp.int32),
            pltpu.VMEM((EB,), jnp.int32),
            pltpu.VMEM((EB,), jnp.int32),
            pltpu.VMEM((EB,), jnp.int32),
            pltpu.VMEM((EB,), jnp.int32),
            pltpu.VMEM((EB,), jnp.int32),
            pltpu.VMEM((EB,), jnp.int32),
            pltpu.VMEM((EB,), jnp.int32),
            pltpu.VMEM((EB, D), jnp.float32),
            pltpu.VMEM((EB, D), jnp.float32),
            pltpu.VMEM((EB, D), jnp.float32),
            pltpu.VMEM((EB, D), jnp.float32),
            pltpu.SemaphoreType.DMA,
            pltpu.SemaphoreType.DMA,
            pltpu.SemaphoreType.DMA,
            pltpu.SemaphoreType.DMA,
            pltpu.SemaphoreType.DMA,
            pltpu.SemaphoreType.DMA,
            pltpu.VMEM_SHARED((NPAD, D), jnp.float32),
        ],
    )


def _split_bf16(a):
    hi = a.astype(jnp.bfloat16)
    lo = (a - hi.astype(jnp.float32)).astype(jnp.bfloat16)
    return hi, lo


_AB = 1000


def _atom_body(x_ref, emb_ref, h_ref):
    col = lax.broadcasted_iota(jnp.int32, (_AB, 128), 1)
    acc = jnp.zeros((_AB, D), jnp.float32)
    for i in range(9):
        oh = (x_ref[:, pl.ds(i, 1)] == col).astype(jnp.bfloat16)
        e = emb_ref[i]
        hi = e.astype(jnp.bfloat16)
        r1 = e - hi.astype(jnp.float32)
        lo = r1.astype(jnp.bfloat16)
        lo2 = (r1 - lo.astype(jnp.float32)).astype(jnp.bfloat16)
        acc = acc + jnp.dot(oh, hi, preferred_element_type=jnp.float32)
        acc = acc + jnp.dot(oh, lo, preferred_element_type=jnp.float32)
        acc = acc + jnp.dot(oh, lo2, preferred_element_type=jnp.float32)
    h_ref[...] = acc


def _atom_encode(x, atom_emb):
    return pl.pallas_call(
        _atom_body,
        grid=(N // _AB,),
        in_specs=[
            pl.BlockSpec((_AB, 9), lambda i: (i, 0)),
            pl.BlockSpec((9, 128, 128), lambda i: (0, 0, 0)),
        ],
        out_specs=pl.BlockSpec((_AB, D), lambda i: (i, 0)),
        out_shape=jax.ShapeDtypeStruct((N, D), jnp.float32),
    )(x, atom_emb)


def _mm1_body(h_ref, pa_ref, pb_ref, eps_ref, w1_ref, b1_ref, y_ref):
    aggr = pa_ref[0:N, :] + pb_ref[0:N, :]
    z = (1.0 + eps_ref[0, 0]) * h_ref[...] + aggr
    y_ref[...] = jnp.dot(z, w1_ref[...], preferred_element_type=jnp.float32) + b1_ref[...]


def _mm2_body(y_ref, g1_ref, bb1_ref, w2_ref, b2_ref, y2_ref):
    y = y_ref[...]
    m = jnp.mean(y, axis=0, keepdims=True)
    yc = y - m
    v = jnp.mean(yc * yc, axis=0, keepdims=True)
    y = yc / jnp.sqrt(v + 1e-5) * g1_ref[...] + bb1_ref[...]
    y = jnp.maximum(y, 0.0)
    y2_ref[...] = jnp.dot(y, w2_ref[...], preferred_element_type=jnp.float32) + b2_ref[...]


def _bn2_body(y2_ref, g2_ref, bb2_ref, batch_ref, h_out_ref, pool_ref, *,
              relu_out):
    y2 = y2_ref[...]
    m2 = jnp.mean(y2, axis=0, keepdims=True)
    yc2 = y2 - m2
    v2 = jnp.mean(yc2 * yc2, axis=0, keepdims=True)
    h2 = yc2 / jnp.sqrt(v2 + 1e-5) * g2_ref[...] + bb2_ref[...]
    if relu_out:
        h2 = jnp.maximum(h2, 0.0)
    h_out_ref[...] = h2
    gi = lax.broadcasted_iota(jnp.int32, (N, G), 1)
    eq = (batch_ref[...] == gi).astype(jnp.bfloat16)
    h2h, h2l = _split_bf16(h2)
    dn = (((0,), (0,)), ((), ()))
    pool_ref[...] = (
        lax.dot_general(eq, h2h, dn, preferred_element_type=jnp.float32)
        + lax.dot_general(eq, h2l, dn, preferred_element_type=jnp.float32))


def _node_update(relu_out):
    mm1 = pl.pallas_call(
        _mm1_body, out_shape=jax.ShapeDtypeStruct((N, D), jnp.float32))
    mm2 = pl.pallas_call(
        _mm2_body, out_shape=jax.ShapeDtypeStruct((N, D), jnp.float32))
    bn2 = pl.pallas_call(
        functools.partial(_bn2_body, relu_out=relu_out),
        out_shape=(
            jax.ShapeDtypeStruct((N, D), jnp.float32),
            jax.ShapeDtypeStruct((G, D), jnp.float32),
        ))

    def run(h, pa, pb, epsl, w1, b1, g1, bb1, w2, b2, g2, bb2, batch2):
        y1 = mm1(h, pa, pb, epsl, w1, b1)
        y2 = mm2(y1, g1, bb1, w2, b2)
        return bn2(y2, g2, bb2, batch2)

    return run


def kernel(x, edge_index, edge_attr, batch, atom_emb, bond_emb, eps,
           W1, b1, bn1_g, bn1_b, W2, b2, bn2_g, bn2_b):
    src = edge_index[0].astype(jnp.int32)
    dst = edge_index[1].astype(jnp.int32)
    ea = edge_attr.astype(jnp.int32)
    cidx = ea[:, 0] * 25 + ea[:, 1] * 5 + ea[:, 2]
    pad = EPAD - E
    src_p = jnp.concatenate([src, jnp.zeros((pad,), jnp.int32)])
    dst_p = jnp.concatenate([dst, jnp.full((pad,), N, jnp.int32)])
    cid_p = jnp.concatenate([cidx, jnp.zeros((pad,), jnp.int32)])
    zeros_hbm = jnp.zeros((NPAD, D), jnp.float32)
    T = (bond_emb[:, 0, :5][:, :, None, None, :]
         + bond_emb[:, 1, :5][:, None, :, None, :]
         + bond_emb[:, 2, :5][:, None, None, :, :]).reshape(L, 125, D)

    h = _atom_encode(x.astype(jnp.int32), atom_emb)
    batch2 = batch.astype(jnp.int32).reshape(N, 1)

    fps = []
    for l in range(L):
        parts = _edge_kernel()(h, T[l], src_p, cid_p, dst_p, zeros_hbm)
        h, pool = _node_update(relu_out=(l < L - 1))(
            h, parts[0], parts[1], eps[l].reshape(1, 1), W1[l],
            b1[l].reshape(1, D), bn1_g[l].reshape(1, D),
            bn1_b[l].reshape(1, D), W2[l], b2[l].reshape(1, D),
            bn2_g[l].reshape(1, D), bn2_b[l].reshape(1, D), batch2)
        fps.append(pool)
    return h, jnp.stack(fps, axis=1)


# speedup vs baseline: 1.3350x; 1.3350x over previous
import functools

import jax
import jax.numpy as jnp
from jax import lax
from jax.experimental import pallas as pl
from jax.experimental.pallas import tpu as pltpu
from jax.experimental.pallas import tpu_sc as plsc

N = 10000
E = 320000
D = 128
L = 3
G = 64

NC = 2
NS = 16
NW = NC * NS
EB = 96
BPT = 106
EPAD = NW * BPT * EB
RPT = 632
NPAD = NS * RPT


def _edge_body(h_hbm, t_hbm, src_hbm, cid_hbm, dst_hbm, zero_hbm, out_hbm,
               isa, isb, ica, icb, idsa, idsb, ida, idb,
               hba, hbb, eba, tbl,
               sia, sib, sga, sgb, ssa, ssb, se, aggr):
    cid = lax.axis_index("c")
    sid = lax.axis_index("s")
    wid = sid * NC + cid
    isx = (isa, isb)
    icx = (ica, icb)
    idsx = (idsa, idsb)
    idd = (ida, idb)
    hb = (hba, hbb)
    ebf = (eba, eba)
    si = (sia, sib)
    sg = (sga, sgb)
    ss = (ssa, ssb)

    pltpu.sync_copy(zero_hbm.at[pl.ds(sid * RPT, RPT)],
                    aggr.at[pl.ds(sid * RPT, RPT)])
    @pl.when(sid == 0)
    def _stage_tbl():
        pltpu.sync_copy(t_hbm, tbl)
    plsc.subcore_barrier()

    tb = wid * BPT

    def issue_idx(b, p):
        base = (tb + b) * EB
        pltpu.async_copy(src_hbm.at[pl.ds(base, EB)], isx[p], si[p])
        pltpu.async_copy(cid_hbm.at[pl.ds(base, EB)], icx[p], si[p])
        pltpu.async_copy(dst_hbm.at[pl.ds(base, EB)], idsx[p], si[p])

    def wait_idx(b, p):
        base = (tb + b) * EB
        pltpu.make_async_copy(src_hbm.at[pl.ds(base, EB)], isx[p], si[p]).wait()
        pltpu.make_async_copy(cid_hbm.at[pl.ds(base, EB)], icx[p], si[p]).wait()
        pltpu.make_async_copy(dst_hbm.at[pl.ds(base, EB)], idsx[p], si[p]).wait()

    def gathers(p):
        pltpu.async_copy(h_hbm.at[isx[p]], hb[p], sg[p])

    def wait_gathers(p):
        pltpu.make_async_copy(h_hbm.at[isx[p]], hb[p], sg[p]).wait()

    def egather(p):
        pltpu.async_copy(tbl.at[icx[p]], ebf[p], se)
        pltpu.make_async_copy(tbl.at[icx[p]], ebf[p], se).wait()

    def copy_dst_idx(p):
        for j in range(EB // 16):
            sl = pl.ds(j * 16, 16)
            idd[p][sl] = idsx[p][sl]

    def compute(p):
        egather(p)
        def row(i, c2):
            for j in range(D // 16):
                sl = pl.ds(j * 16, 16)
                hb[p][i, sl] = jnp.maximum(hb[p][i, sl] + ebf[p][i, sl], 0.0)
            return c2
        lax.fori_loop(0, EB, row, 0)

    def scatter(p):
        pltpu.async_copy(hb[p], aggr.at[idd[p]], ss[p], add=True)

    def wait_scatter(p):
        pltpu.make_async_copy(hb[p], aggr.at[idd[p]], ss[p]).wait()

    issue_idx(0, 0)
    issue_idx(1, 1)
    wait_idx(0, 0)
    gathers(0)

    wait_gathers(0)
    copy_dst_idx(0)
    wait_idx(1, 1)
    gathers(1)
    issue_idx(2, 0)
    compute(0)
    scatter(0)

    wait_gathers(1)
    copy_dst_idx(1)
    wait_idx(2, 0)
    wait_scatter(0)
    gathers(0)
    issue_idx(3, 1)
    compute(1)
    scatter(1)

    def pair(j, carry):
        for k in range(2):
            b = 2 * j + k
            q = 1 - k
            wait_gathers(k)
            copy_dst_idx(k)
            wait_idx(b + 1, q)
            wait_scatter(q)
            gathers(q)
            issue_idx(b + 2, k)
            compute(k)
            scatter(k)
        return carry

    lax.fori_loop(1, BPT // 2 - 1, pair, 0)

    wait_gathers(0)
    copy_dst_idx(0)
    wait_idx(BPT - 1, 1)
    wait_scatter(1)
    gathers(1)
    compute(0)
    scatter(0)

    wait_gathers(1)
    copy_dst_idx(1)
    compute(1)
    wait_scatter(0)
    scatter(1)
    wait_scatter(1)

    plsc.subcore_barrier()
    pltpu.sync_copy(aggr.at[pl.ds(sid * RPT, RPT)],
                    out_hbm.at[cid, pl.ds(sid * RPT, RPT)])


@functools.lru_cache(maxsize=None)
def _edge_kernel():
    return _edge_wrap(_edge_body)


def _edge_wrap(body):
    return pl.kernel(
        body,
        out_type=jax.ShapeDtypeStruct((NC, NPAD, D), jnp.float32),
        mesh=plsc.VectorSubcoreMesh(core_axis_name="c", subcore_axis_name="s"),
        scratch_types=[
            pltpu.VMEM((EB,), jnp.int32),
            pltpu.VMEM((EB,), jnp.int32),
            pltpu.VMEM((EB,), jnp.int32),
            pltpu.VMEM((EB,), jnp.int32),
            pltpu.VMEM((EB,), jnp.int32),
            pltpu.VMEM((EB,), jnp.int32),
            pltpu.VMEM((EB,), jnp.int32),
            pltpu.VMEM((EB,), jnp.int32),
            pltpu.VMEM((EB, D), jnp.float32),
            pltpu.VMEM((EB, D), jnp.float32),
            pltpu.VMEM((EB, D), jnp.float32),
            pltpu.VMEM_SHARED((125, D), jnp.float32),
            pltpu.SemaphoreType.DMA,
            pltpu.SemaphoreType.DMA,
            pltpu.SemaphoreType.DMA,
            pltpu.SemaphoreType.DMA,
            pltpu.SemaphoreType.DMA,
            pltpu.SemaphoreType.DMA,
            pltpu.SemaphoreType.DMA,
            pltpu.VMEM_SHARED((NPAD, D), jnp.float32),
        ],
    )


def _split_bf16(a):
    hi = a.astype(jnp.bfloat16)
    lo = (a - hi.astype(jnp.float32)).astype(jnp.bfloat16)
    return hi, lo


_AB = 1000


def _atom_body(x_ref, emb_ref, h_ref):
    col = lax.broadcasted_iota(jnp.int32, (_AB, 128), 1)
    acc = jnp.zeros((_AB, D), jnp.float32)
    for i in range(9):
        oh = (x_ref[:, pl.ds(i, 1)] == col).astype(jnp.bfloat16)
        e = emb_ref[i]
        hi = e.astype(jnp.bfloat16)
        r1 = e - hi.astype(jnp.float32)
        lo = r1.astype(jnp.bfloat16)
        lo2 = (r1 - lo.astype(jnp.float32)).astype(jnp.bfloat16)
        acc = acc + jnp.dot(oh, hi, preferred_element_type=jnp.float32)
        acc = acc + jnp.dot(oh, lo, preferred_element_type=jnp.float32)
        acc = acc + jnp.dot(oh, lo2, preferred_element_type=jnp.float32)
    h_ref[...] = acc


def _atom_encode(x, atom_emb):
    return pl.pallas_call(
        _atom_body,
        grid=(N // _AB,),
        in_specs=[
            pl.BlockSpec((_AB, 9), lambda i: (i, 0)),
            pl.BlockSpec((9, 128, 128), lambda i: (0, 0, 0)),
        ],
        out_specs=pl.BlockSpec((_AB, D), lambda i: (i, 0)),
        out_shape=jax.ShapeDtypeStruct((N, D), jnp.float32),
    )(x, atom_emb)


def _mm1_body(h_ref, pa_ref, pb_ref, eps_ref, w1_ref, b1_ref, y_ref):
    aggr = pa_ref[0:N, :] + pb_ref[0:N, :]
    z = (1.0 + eps_ref[0, 0]) * h_ref[...] + aggr
    y_ref[...] = jnp.dot(z, w1_ref[...], preferred_element_type=jnp.float32) + b1_ref[...]


def _mm2_body(y_ref, g1_ref, bb1_ref, w2_ref, b2_ref, y2_ref):
    y = y_ref[...]
    m = jnp.mean(y, axis=0, keepdims=True)
    yc = y - m
    v = jnp.mean(yc * yc, axis=0, keepdims=True)
    y = yc / jnp.sqrt(v + 1e-5) * g1_ref[...] + bb1_ref[...]
    y = jnp.maximum(y, 0.0)
    y2_ref[...] = jnp.dot(y, w2_ref[...], preferred_element_type=jnp.float32) + b2_ref[...]


def _bn2_body(y2_ref, g2_ref, bb2_ref, batch_ref, h_out_ref, pool_ref, *,
              relu_out):
    y2 = y2_ref[...]
    m2 = jnp.mean(y2, axis=0, keepdims=True)
    yc2 = y2 - m2
    v2 = jnp.mean(yc2 * yc2, axis=0, keepdims=True)
    h2 = yc2 / jnp.sqrt(v2 + 1e-5) * g2_ref[...] + bb2_ref[...]
    if relu_out:
        h2 = jnp.maximum(h2, 0.0)
    h_out_ref[...] = h2
    gi = lax.broadcasted_iota(jnp.int32, (N, G), 1)
    eq = (batch_ref[...] == gi).astype(jnp.bfloat16)
    h2h, h2l = _split_bf16(h2)
    dn = (((0,), (0,)), ((), ()))
    pool_ref[...] = (
        lax.dot_general(eq, h2h, dn, preferred_element_type=jnp.float32)
        + lax.dot_general(eq, h2l, dn, preferred_element_type=jnp.float32))


def _node_update(relu_out):
    mm1 = pl.pallas_call(
        _mm1_body, out_shape=jax.ShapeDtypeStruct((N, D), jnp.float32))
    mm2 = pl.pallas_call(
        _mm2_body, out_shape=jax.ShapeDtypeStruct((N, D), jnp.float32))
    bn2 = pl.pallas_call(
        functools.partial(_bn2_body, relu_out=relu_out),
        out_shape=(
            jax.ShapeDtypeStruct((N, D), jnp.float32),
            jax.ShapeDtypeStruct((G, D), jnp.float32),
        ))

    def run(h, pa, pb, epsl, w1, b1, g1, bb1, w2, b2, g2, bb2, batch2):
        y1 = mm1(h, pa, pb, epsl, w1, b1)
        y2 = mm2(y1, g1, bb1, w2, b2)
        return bn2(y2, g2, bb2, batch2)

    return run


def kernel(x, edge_index, edge_attr, batch, atom_emb, bond_emb, eps,
           W1, b1, bn1_g, bn1_b, W2, b2, bn2_g, bn2_b):
    src = edge_index[0].astype(jnp.int32)
    dst = edge_index[1].astype(jnp.int32)
    ea = edge_attr.astype(jnp.int32)
    cidx = ea[:, 0] * 25 + ea[:, 1] * 5 + ea[:, 2]
    pad = EPAD - E
    src_p = jnp.concatenate([src, jnp.zeros((pad,), jnp.int32)])
    dst_p = jnp.concatenate([dst, jnp.full((pad,), N, jnp.int32)])
    cid_p = jnp.concatenate([cidx, jnp.zeros((pad,), jnp.int32)])
    zeros_hbm = jnp.zeros((NPAD, D), jnp.float32)
    T = (bond_emb[:, 0, :5][:, :, None, None, :]
         + bond_emb[:, 1, :5][:, None, :, None, :]
         + bond_emb[:, 2, :5][:, None, None, :, :]).reshape(L, 125, D)

    h = _atom_encode(x.astype(jnp.int32), atom_emb)
    batch2 = batch.astype(jnp.int32).reshape(N, 1)

    fps = []
    for l in range(L):
        parts = _edge_kernel()(h, T[l], src_p, cid_p, dst_p, zeros_hbm)
        h, pool = _node_update(relu_out=(l < L - 1))(
            h, parts[0], parts[1], eps[l].reshape(1, 1), W1[l],
            b1[l].reshape(1, D), bn1_g[l].reshape(1, D),
            bn1_b[l].reshape(1, D), W2[l], b2[l].reshape(1, D),
            bn2_g[l].reshape(1, D), bn2_b[l].reshape(1, D), batch2)
        fps.append(pool)
    return h, jnp.stack(fps, axis=1)


# speedup vs baseline: 1.3352x; 1.0002x over previous
import functools

import jax
import jax.numpy as jnp
from jax import lax
from jax.experimental import pallas as pl
from jax.experimental.pallas import tpu as pltpu
from jax.experimental.pallas import tpu_sc as plsc

N = 10000
E = 320000
D = 128
L = 3
G = 64

NC = 2
NS = 16
NW = NC * NS
EB = 96
BPT = 106
EPAD = NW * BPT * EB
RPT = 632
NPAD = NS * RPT


def _edge_body(h_hbm, t_hbm, src_hbm, cid_hbm, dst_hbm, zero_hbm, out_hbm,
               isa, isb, ica, icb, idsa, idsb, ida, idb,
               hba, hbb, eba, tbl,
               sia, sib, sga, sgb, ssa, ssb, sea, aggr):
    cid = lax.axis_index("c")
    sid = lax.axis_index("s")
    wid = sid * NC + cid
    isx = (isa, isb)
    icx = (ica, icb)
    idsx = (idsa, idsb)
    idd = (ida, idb)
    hb = (hba, hbb)
    ebf = (eba, eba)
    se = (sea, sea)
    si = (sia, sib)
    sg = (sga, sgb)
    ss = (ssa, ssb)

    pltpu.sync_copy(zero_hbm.at[pl.ds(sid * RPT, RPT)],
                    aggr.at[pl.ds(sid * RPT, RPT)])
    @pl.when(sid == 0)
    def _stage_tbl():
        pltpu.sync_copy(t_hbm, tbl)
    plsc.subcore_barrier()

    tb = wid * BPT

    def issue_idx(b, p):
        base = (tb + b) * EB
        pltpu.async_copy(src_hbm.at[pl.ds(base, EB)], isx[p], si[p])
        pltpu.async_copy(cid_hbm.at[pl.ds(base, EB)], icx[p], si[p])
        pltpu.async_copy(dst_hbm.at[pl.ds(base, EB)], idsx[p], si[p])

    def wait_idx(b, p):
        base = (tb + b) * EB
        pltpu.make_async_copy(src_hbm.at[pl.ds(base, EB)], isx[p], si[p]).wait()
        pltpu.make_async_copy(cid_hbm.at[pl.ds(base, EB)], icx[p], si[p]).wait()
        pltpu.make_async_copy(dst_hbm.at[pl.ds(base, EB)], idsx[p], si[p]).wait()

    def gathers(p):
        pltpu.async_copy(h_hbm.at[isx[p]], hb[p], sg[p])

    def wait_gathers(p):
        pltpu.make_async_copy(h_hbm.at[isx[p]], hb[p], sg[p]).wait()

    def egather(p):
        pltpu.async_copy(tbl.at[icx[p]], ebf[p], se[p])
        pltpu.make_async_copy(tbl.at[icx[p]], ebf[p], se[p]).wait()

    def copy_dst_idx(p):
        for j in range(EB // 16):
            sl = pl.ds(j * 16, 16)
            idd[p][sl] = idsx[p][sl]

    def compute(p):
        egather(p)
        def row(i, c2):
            for j in range(D // 16):
                sl = pl.ds(j * 16, 16)
                hb[p][i, sl] = jnp.maximum(hb[p][i, sl] + ebf[p][i, sl], 0.0)
            return c2
        lax.fori_loop(0, EB, row, 0)

    def scatter(p):
        pltpu.async_copy(hb[p], aggr.at[idd[p]], ss[p], add=True)

    def wait_scatter(p):
        pltpu.make_async_copy(hb[p], aggr.at[idd[p]], ss[p]).wait()

    issue_idx(0, 0)
    issue_idx(1, 1)
    wait_idx(0, 0)
    gathers(0)

    wait_gathers(0)
    copy_dst_idx(0)
    wait_idx(1, 1)
    gathers(1)
    compute(0)
    issue_idx(2, 0)
    scatter(0)

    wait_gathers(1)
    copy_dst_idx(1)
    wait_idx(2, 0)
    wait_scatter(0)
    gathers(0)
    compute(1)
    issue_idx(3, 1)
    scatter(1)

    def pair(j, carry):
        for k in range(2):
            b = 2 * j + k
            q = 1 - k
            wait_gathers(k)
            copy_dst_idx(k)
            wait_idx(b + 1, q)
            wait_scatter(q)
            gathers(q)
            compute(k)
            issue_idx(b + 2, k)
            scatter(k)
        return carry

    lax.fori_loop(1, BPT // 2 - 1, pair, 0)

    wait_gathers(0)
    copy_dst_idx(0)
    wait_idx(BPT - 1, 1)
    wait_scatter(1)
    gathers(1)
    compute(0)
    scatter(0)

    wait_gathers(1)
    copy_dst_idx(1)
    compute(1)
    wait_scatter(0)
    scatter(1)
    wait_scatter(1)

    plsc.subcore_barrier()
    pltpu.sync_copy(aggr.at[pl.ds(sid * RPT, RPT)],
                    out_hbm.at[cid, pl.ds(sid * RPT, RPT)])


@functools.lru_cache(maxsize=None)
def _edge_kernel():
    return _edge_wrap(_edge_body)


def _edge_wrap(body):
    return pl.kernel(
        body,
        out_type=jax.ShapeDtypeStruct((NC, NPAD, D), jnp.float32),
        mesh=plsc.VectorSubcoreMesh(core_axis_name="c", subcore_axis_name="s"),
        scratch_types=[
            pltpu.VMEM((EB,), jnp.int32),
            pltpu.VMEM((EB,), jnp.int32),
            pltpu.VMEM((EB,), jnp.int32),
            pltpu.VMEM((EB,), jnp.int32),
            pltpu.VMEM((EB,), jnp.int32),
            pltpu.VMEM((EB,), jnp.int32),
            pltpu.VMEM((EB,), jnp.int32),
            pltpu.VMEM((EB,), jnp.int32),
            pltpu.VMEM((EB, D), jnp.float32),
            pltpu.VMEM((EB, D), jnp.float32),
            pltpu.VMEM((EB, D), jnp.float32),
            pltpu.VMEM_SHARED((125, D), jnp.float32),
            pltpu.SemaphoreType.DMA,
            pltpu.SemaphoreType.DMA,
            pltpu.SemaphoreType.DMA,
            pltpu.SemaphoreType.DMA,
            pltpu.SemaphoreType.DMA,
            pltpu.SemaphoreType.DMA,
            pltpu.SemaphoreType.DMA,
            pltpu.VMEM_SHARED((NPAD, D), jnp.float32),
        ],
    )


def _split_bf16(a):
    hi = a.astype(jnp.bfloat16)
    lo = (a - hi.astype(jnp.float32)).astype(jnp.bfloat16)
    return hi, lo


_AB = 1000


def _atom_body(x_ref, emb_ref, h_ref):
    col = lax.broadcasted_iota(jnp.int32, (_AB, 128), 1)
    acc = jnp.zeros((_AB, D), jnp.float32)
    for i in range(9):
        oh = (x_ref[:, pl.ds(i, 1)] == col).astype(jnp.bfloat16)
        e = emb_ref[i]
        hi = e.astype(jnp.bfloat16)
        r1 = e - hi.astype(jnp.float32)
        lo = r1.astype(jnp.bfloat16)
        lo2 = (r1 - lo.astype(jnp.float32)).astype(jnp.bfloat16)
        acc = acc + jnp.dot(oh, hi, preferred_element_type=jnp.float32)
        acc = acc + jnp.dot(oh, lo, preferred_element_type=jnp.float32)
        acc = acc + jnp.dot(oh, lo2, preferred_element_type=jnp.float32)
    h_ref[...] = acc


def _atom_encode(x, atom_emb):
    return pl.pallas_call(
        _atom_body,
        grid=(N // _AB,),
        in_specs=[
            pl.BlockSpec((_AB, 9), lambda i: (i, 0)),
            pl.BlockSpec((9, 128, 128), lambda i: (0, 0, 0)),
        ],
        out_specs=pl.BlockSpec((_AB, D), lambda i: (i, 0)),
        out_shape=jax.ShapeDtypeStruct((N, D), jnp.float32),
    )(x, atom_emb)


def _mm1_body(h_ref, pa_ref, pb_ref, eps_ref, w1_ref, b1_ref, y_ref):
    aggr = pa_ref[0:N, :] + pb_ref[0:N, :]
    z = (1.0 + eps_ref[0, 0]) * h_ref[...] + aggr
    y_ref[...] = jnp.dot(z, w1_ref[...], preferred_element_type=jnp.float32) + b1_ref[...]


def _mm2_body(y_ref, g1_ref, bb1_ref, w2_ref, b2_ref, y2_ref):
    y = y_ref[...]
    m = jnp.mean(y, axis=0, keepdims=True)
    yc = y - m
    v = jnp.mean(yc * yc, axis=0, keepdims=True)
    y = yc / jnp.sqrt(v + 1e-5) * g1_ref[...] + bb1_ref[...]
    y = jnp.maximum(y, 0.0)
    y2_ref[...] = jnp.dot(y, w2_ref[...], preferred_element_type=jnp.float32) + b2_ref[...]


def _bn2_body(y2_ref, g2_ref, bb2_ref, batch_ref, h_out_ref, pool_ref, *,
              relu_out):
    y2 = y2_ref[...]
    m2 = jnp.mean(y2, axis=0, keepdims=True)
    yc2 = y2 - m2
    v2 = jnp.mean(yc2 * yc2, axis=0, keepdims=True)
    h2 = yc2 / jnp.sqrt(v2 + 1e-5) * g2_ref[...] + bb2_ref[...]
    if relu_out:
        h2 = jnp.maximum(h2, 0.0)
    h_out_ref[...] = h2
    gi = lax.broadcasted_iota(jnp.int32, (N, G), 1)
    eq = (batch_ref[...] == gi).astype(jnp.bfloat16)
    h2h, h2l = _split_bf16(h2)
    dn = (((0,), (0,)), ((), ()))
    pool_ref[...] = (
        lax.dot_general(eq, h2h, dn, preferred_element_type=jnp.float32)
        + lax.dot_general(eq, h2l, dn, preferred_element_type=jnp.float32))


def _node_update(relu_out):
    mm1 = pl.pallas_call(
        _mm1_body, out_shape=jax.ShapeDtypeStruct((N, D), jnp.float32))
    mm2 = pl.pallas_call(
        _mm2_body, out_shape=jax.ShapeDtypeStruct((N, D), jnp.float32))
    bn2 = pl.pallas_call(
        functools.partial(_bn2_body, relu_out=relu_out),
        out_shape=(
            jax.ShapeDtypeStruct((N, D), jnp.float32),
            jax.ShapeDtypeStruct((G, D), jnp.float32),
        ))

    def run(h, pa, pb, epsl, w1, b1, g1, bb1, w2, b2, g2, bb2, batch2):
        y1 = mm1(h, pa, pb, epsl, w1, b1)
        y2 = mm2(y1, g1, bb1, w2, b2)
        return bn2(y2, g2, bb2, batch2)

    return run


def kernel(x, edge_index, edge_attr, batch, atom_emb, bond_emb, eps,
           W1, b1, bn1_g, bn1_b, W2, b2, bn2_g, bn2_b):
    src = edge_index[0].astype(jnp.int32)
    dst = edge_index[1].astype(jnp.int32)
    ea = edge_attr.astype(jnp.int32)
    cidx = ea[:, 0] * 25 + ea[:, 1] * 5 + ea[:, 2]
    pad = EPAD - E
    src_p = jnp.concatenate([src, jnp.zeros((pad,), jnp.int32)])
    dst_p = jnp.concatenate([dst, jnp.full((pad,), N, jnp.int32)])
    cid_p = jnp.concatenate([cidx, jnp.zeros((pad,), jnp.int32)])
    zeros_hbm = jnp.zeros((NPAD, D), jnp.float32)
    T = (bond_emb[:, 0, :5][:, :, None, None, :]
         + bond_emb[:, 1, :5][:, None, :, None, :]
         + bond_emb[:, 2, :5][:, None, None, :, :]).reshape(L, 125, D)

    h = _atom_encode(x.astype(jnp.int32), atom_emb)
    batch2 = batch.astype(jnp.int32).reshape(N, 1)

    fps = []
    for l in range(L):
        parts = _edge_kernel()(h, T[l], src_p, cid_p, dst_p, zeros_hbm)
        h, pool = _node_update(relu_out=(l < L - 1))(
            h, parts[0], parts[1], eps[l].reshape(1, 1), W1[l],
            b1[l].reshape(1, D), bn1_g[l].reshape(1, D),
            bn1_b[l].reshape(1, D), W2[l], b2[l].reshape(1, D),
            bn2_g[l].reshape(1, D), bn2_b[l].reshape(1, D), batch2)
        fps.append(pool)
    return h, jnp.stack(fps, axis=1)
